# baseline (speedup 1.0000x reference)
import functools

import jax
import jax.numpy as jnp
from jax import lax
from jax.experimental import pallas as pl
from jax.experimental.pallas import tpu as pltpu
from jax.experimental.pallas import tpu_sc as plsc

VOCAB = 1000000
SEQ_LEN = 200
DIM = 64
BATCH = 4096

NUM_CORES = 2
NUM_SUBCORES = 16
NUM_WORKERS = NUM_CORES * NUM_SUBCORES
NCHUNKS = BATCH // NUM_WORKERS
NBUF = 4


def _fire_idx(tok_hbm, idxv, sem, seq):
    pltpu.async_copy(tok_hbm.at[pl.ds(seq, 1)], idxv, sem)


def _wait_idx(tok_hbm, idxv, sem, seq):
    pltpu.make_async_copy(tok_hbm.at[pl.ds(seq, 1)], idxv, sem).wait()


def _enqueue_rows(table_hbm, gbuf, sem, r0, lanes, toks):
    for lane in lanes:
        tok = toks[lane]
        pltpu.async_copy(table_hbm.at[pl.ds(tok, 1)],
                         gbuf.at[pl.ds(r0 + lane, 1)], sem)


def _fire_gather(table_hbm, idxv, gbuf, sem):
    def row_group(g, _):
        toks = idxv[0, pl.ds(g * 16, 16)]
        _enqueue_rows(table_hbm, gbuf, sem, g * 16, range(16), toks)
        return _
    lax.fori_loop(0, SEQ_LEN // 16, row_group, None)
    toks_t = idxv[0, pl.ds(SEQ_LEN - 16, 16)]
    _enqueue_rows(table_hbm, gbuf, sem, SEQ_LEN - 16, range(8, 16), toks_t)


def _wait_gather(table_hbm, gbuf, sem):
    pltpu.make_async_copy(table_hbm.at[pl.ds(0, SEQ_LEN)], gbuf, sem).wait()


def _fire_store(gbuf, out_hbm, sem, seq):
    pltpu.async_copy(gbuf, out_hbm.at[pl.ds(seq * SEQ_LEN, SEQ_LEN)], sem)


def _wait_store(gbuf, out_hbm, sem, seq):
    pltpu.make_async_copy(
        gbuf, out_hbm.at[pl.ds(seq * SEQ_LEN, SEQ_LEN)], sem).wait()


def _add_pe(gbuf, pe_v):
    @plsc.parallel_loop(0, SEQ_LEN // 2, step=1, unroll=4)
    def _(r2):
        r = r2 * 2
        for j in range(DIM // 16):
            sl = pl.ds(j * 16, 16)
            gbuf[r, sl] = gbuf[r, sl] + pe_v[r2, sl]
            gbuf[r + 1, sl] = gbuf[r + 1, sl] + pe_v[r2, pl.ds(DIM + j * 16, 16)]


@functools.partial(
    pl.kernel,
    mesh=plsc.VectorSubcoreMesh(core_axis_name="c", subcore_axis_name="s"),
    out_type=jax.ShapeDtypeStruct((BATCH * SEQ_LEN, DIM), jnp.float32),
    compiler_params=pltpu.CompilerParams(use_tc_tiling_on_sc=True),
    scratch_types=(
        [pltpu.VMEM((SEQ_LEN // 2, 2 * DIM), jnp.float32)]
        + [pltpu.VMEM((SEQ_LEN, DIM), jnp.float32)] * NBUF
        + [pltpu.VMEM((1, SEQ_LEN), jnp.int32)] * NBUF
        + [pltpu.SemaphoreType.DMA] * (3 * NBUF)
    ),
)
def _embed_kernel(tok_hbm, table_hbm, pe_hbm, out_hbm, pe_v,
                  b0, b1, b2, b3, m0, m1, m2, m3,
                  si0, si1, si2, si3, g0, g1, g2, g3, s0, s1, s2, s3):
    gbufs = [b0, b1, b2, b3]
    idxvs = [m0, m1, m2, m3]
    isems = [si0, si1, si2, si3]
    gsems = [g0, g1, g2, g3]
    ssems = [s0, s1, s2, s3]
    wid = lax.axis_index("s") * NUM_CORES + lax.axis_index("c")
    seq0 = wid * NCHUNKS

    pltpu.sync_copy(pe_hbm, pe_v)

    def fire_idx(c, b):
        _fire_idx(tok_hbm, idxvs[b], isems[b], seq0 + c)

    def fire_gather(c, b):
        _wait_idx(tok_hbm, idxvs[b], isems[b], seq0 + c)
        _fire_gather(table_hbm, idxvs[b], gbufs[b], gsems[b])

    def step(c, b, do_store_wait, do_fire_idx, do_fire_gather):
        b2 = (b + 2) % NBUF
        b3 = (b + 3) % NBUF
        if do_store_wait:
            _wait_store(gbufs[b2], out_hbm, ssems[b2], seq0 + c - 2)
        if do_fire_idx:
            fire_idx(c + 3, b3)
        if do_fire_gather:
            fire_gather(c + 2, b2)
        _wait_gather(table_hbm, gbufs[b], gsems[b])
        _add_pe(gbufs[b], pe_v)
        _fire_store(gbufs[b], out_hbm, ssems[b], seq0 + c)

    fire_idx(0, 0)
    fire_idx(1, 1)
    fire_idx(2, 2)
    fire_gather(0, 0)
    fire_gather(1, 1)

    for b in range(NBUF):
        step(jnp.int32(b), b, do_store_wait=(b >= 2),
             do_fire_idx=True, do_fire_gather=True)

    def group(g, _):
        for b in range(NBUF):
            step(g * NBUF + b, b, do_store_wait=True,
                 do_fire_idx=True, do_fire_gather=True)
        return _

    lax.fori_loop(1, NCHUNKS // NBUF - 1, group, None)

    gt = NCHUNKS - NBUF
    for b in range(NBUF):
        step(jnp.int32(gt + b), b, do_store_wait=True,
             do_fire_idx=(b < 1), do_fire_gather=(b < 2))

    _wait_store(gbufs[2], out_hbm, ssems[2], seq0 + NCHUNKS - 2)
    _wait_store(gbufs[3], out_hbm, ssems[3], seq0 + NCHUNKS - 1)


def kernel(tokenize, table, pe):
    pe2 = pe.reshape(SEQ_LEN // 2, 2 * DIM)
    out = _embed_kernel(tokenize.astype(jnp.int32), table, pe2)
    return out.reshape(BATCH, SEQ_LEN, DIM)


# speedup vs baseline: 1.0007x; 1.0007x over previous
import functools

import jax
import jax.numpy as jnp
from jax import lax
from jax.experimental import pallas as pl
from jax.experimental.pallas import tpu as pltpu
from jax.experimental.pallas import tpu_sc as plsc

VOCAB = 1000000
SEQ_LEN = 200
DIM = 64
BATCH = 4096

NUM_CORES = 2
NUM_SUBCORES = 16
NUM_WORKERS = NUM_CORES * NUM_SUBCORES
NCHUNKS = BATCH // NUM_WORKERS
NBUF = 4


def _fire_idx(tok_hbm, idxv, sem, seq):
    pltpu.async_copy(tok_hbm.at[pl.ds(seq, 1)], idxv, sem)


def _wait_idx(tok_hbm, idxv, sem, seq):
    pltpu.make_async_copy(tok_hbm.at[pl.ds(seq, 1)], idxv, sem).wait()


def _enqueue_rows(table_hbm, gbuf, sem, r0, lanes, toks):
    for lane in lanes:
        tok = toks[lane]
        pltpu.async_copy(table_hbm.at[pl.ds(tok, 1)],
                         gbuf.at[pl.ds(r0 + lane, 1)], sem)


def _fire_gather(table_hbm, idxv, gbuf, sem):
    def row_group(g, _):
        toks = idxv[0, pl.ds(g * 16, 16)]
        _enqueue_rows(table_hbm, gbuf, sem, g * 16, range(16), toks)
        return _
    lax.fori_loop(0, SEQ_LEN // 16, row_group, None, unroll=2)
    toks_t = idxv[0, pl.ds(SEQ_LEN - 16, 16)]
    _enqueue_rows(table_hbm, gbuf, sem, SEQ_LEN - 16, range(8, 16), toks_t)


def _wait_gather(table_hbm, gbuf, sem):
    pltpu.make_async_copy(table_hbm.at[pl.ds(0, SEQ_LEN)], gbuf, sem).wait()


def _fire_store(gbuf, out_hbm, sem, seq):
    pltpu.async_copy(gbuf, out_hbm.at[pl.ds(seq * SEQ_LEN, SEQ_LEN)], sem)


def _wait_store(gbuf, out_hbm, sem, seq):
    pltpu.make_async_copy(
        gbuf, out_hbm.at[pl.ds(seq * SEQ_LEN, SEQ_LEN)], sem).wait()


def _add_pe(gbuf, pe_v):
    @plsc.parallel_loop(0, SEQ_LEN // 2, step=1, unroll=4)
    def _(r2):
        r = r2 * 2
        for j in range(DIM // 16):
            sl = pl.ds(j * 16, 16)
            gbuf[r, sl] = gbuf[r, sl] + pe_v[r2, sl]
            gbuf[r + 1, sl] = gbuf[r + 1, sl] + pe_v[r2, pl.ds(DIM + j * 16, 16)]


@functools.partial(
    pl.kernel,
    mesh=plsc.VectorSubcoreMesh(core_axis_name="c", subcore_axis_name="s"),
    out_type=jax.ShapeDtypeStruct((BATCH * SEQ_LEN, DIM), jnp.float32),
    compiler_params=pltpu.CompilerParams(use_tc_tiling_on_sc=True),
    scratch_types=(
        [pltpu.VMEM((SEQ_LEN // 2, 2 * DIM), jnp.float32)]
        + [pltpu.VMEM((SEQ_LEN, DIM), jnp.float32)] * NBUF
        + [pltpu.VMEM((1, SEQ_LEN), jnp.int32)] * NBUF
        + [pltpu.SemaphoreType.DMA] * (3 * NBUF)
    ),
)
def _embed_kernel(tok_hbm, table_hbm, pe_hbm, out_hbm, pe_v,
                  b0, b1, b2, b3, m0, m1, m2, m3,
                  si0, si1, si2, si3, g0, g1, g2, g3, s0, s1, s2, s3):
    gbufs = [b0, b1, b2, b3]
    idxvs = [m0, m1, m2, m3]
    isems = [si0, si1, si2, si3]
    gsems = [g0, g1, g2, g3]
    ssems = [s0, s1, s2, s3]
    wid = lax.axis_index("s") * NUM_CORES + lax.axis_index("c")
    seq0 = wid * NCHUNKS

    pltpu.sync_copy(pe_hbm, pe_v)

    def fire_idx(c, b):
        _fire_idx(tok_hbm, idxvs[b], isems[b], seq0 + c)

    def fire_gather(c, b):
        _wait_idx(tok_hbm, idxvs[b], isems[b], seq0 + c)
        _fire_gather(table_hbm, idxvs[b], gbufs[b], gsems[b])

    def step(c, b, do_store_wait, do_fire_idx, do_fire_gather):
        b2 = (b + 2) % NBUF
        b3 = (b + 3) % NBUF
        if do_store_wait:
            _wait_store(gbufs[b2], out_hbm, ssems[b2], seq0 + c - 2)
        if do_fire_idx:
            fire_idx(c + 3, b3)
        if do_fire_gather:
            fire_gather(c + 2, b2)
        _wait_gather(table_hbm, gbufs[b], gsems[b])
        _add_pe(gbufs[b], pe_v)
        _fire_store(gbufs[b], out_hbm, ssems[b], seq0 + c)

    fire_idx(0, 0)
    fire_idx(1, 1)
    fire_idx(2, 2)
    fire_gather(0, 0)
    fire_gather(1, 1)

    for b in range(NBUF):
        step(jnp.int32(b), b, do_store_wait=(b >= 2),
             do_fire_idx=True, do_fire_gather=True)

    def group(g, _):
        for b in range(NBUF):
            step(g * NBUF + b, b, do_store_wait=True,
                 do_fire_idx=True, do_fire_gather=True)
        return _

    lax.fori_loop(1, NCHUNKS // NBUF - 1, group, None)

    gt = NCHUNKS - NBUF
    for b in range(NBUF):
        step(jnp.int32(gt + b), b, do_store_wait=True,
             do_fire_idx=(b < 1), do_fire_gather=(b < 2))

    _wait_store(gbufs[2], out_hbm, ssems[2], seq0 + NCHUNKS - 2)
    _wait_store(gbufs[3], out_hbm, ssems[3], seq0 + NCHUNKS - 1)


def kernel(tokenize, table, pe):
    pe2 = pe.reshape(SEQ_LEN // 2, 2 * DIM)
    out = _embed_kernel(tokenize.astype(jnp.int32), table, pe2)
    return out.reshape(BATCH, SEQ_LEN, DIM)
